# baseline (speedup 1.0000x reference)
import functools

import jax
import jax.numpy as jnp
from jax import lax
from jax.experimental import pallas as pl
from jax.experimental.pallas import tpu as pltpu
from jax.experimental.pallas import tpu_sc as plsc

_NCHUNK = 8


def _dma_copy_body(x_hbm, o_hbm, *sems):
    r = x_hbm.shape[0]
    ch = r // _NCHUNK
    copies = [
        pltpu.make_async_copy(
            x_hbm.at[pl.ds(i * ch, ch)], o_hbm.at[pl.ds(i * ch, ch)], sems[i]
        )
        for i in range(_NCHUNK)
    ]
    for c in copies:
        c.start()
    for c in copies:
        c.wait()


def _tc_copy(x2d):
    return pl.pallas_call(
        _dma_copy_body,
        in_specs=[pl.BlockSpec(memory_space=pl.ANY)],
        out_specs=pl.BlockSpec(memory_space=pl.ANY),
        scratch_shapes=[pltpu.SemaphoreType.DMA] * _NCHUNK,
        out_shape=jax.ShapeDtypeStruct(x2d.shape, x2d.dtype),
    )(x2d)


def _make_sc_scatter(b, s, d, n, chunk=16):
    nc, ns = 2, 16
    mesh = plsc.VectorSubcoreMesh(
        core_axis_name="c", subcore_axis_name="s", num_cores=nc, num_subcores=ns
    )
    nworkers = (b * n) // chunk

    @functools.partial(
        pl.kernel,
        out_type=(),
        mesh=mesh,
        scratch_types=[
            pltpu.VMEM((chunk,), jnp.int32),
            pltpu.VMEM((chunk, d), jnp.float32),
            pltpu.SemaphoreType.DMA,
        ],
    )
    def sc_scatter(out_ref, vals_hbm, idx_hbm, idx_v, rows_v, sem):
        wid = lax.axis_index("s") * nc + lax.axis_index("c")

        @pl.when(wid < nworkers)
        def _():
            t0 = wid * chunk
            batch = t0 // n
            i0 = t0 % n
            pltpu.sync_copy(idx_hbm.at[pl.ds(i0, chunk)], idx_v)
            pltpu.sync_copy(vals_hbm.at[pl.ds(i0, chunk)], rows_v)
            flat = idx_v[...] + batch * s
            pltpu.async_copy(rows_v, out_ref.at[flat], sem).wait()

    return sc_scatter


def kernel(x, replace_vals, replace_idx):
    b, s, d = x.shape
    n = replace_vals.shape[0]
    x2d = x.reshape(b * s, d)
    y = _tc_copy(x2d)
    y_ref = jax.new_ref(y)
    _make_sc_scatter(b, s, d, n)(y_ref, replace_vals, replace_idx)
    return jax.freeze(y_ref).reshape(b, s, d)


# speedup vs baseline: 38.8372x; 38.8372x over previous
import functools

import jax
import jax.numpy as jnp
from jax import lax
from jax.experimental import pallas as pl
from jax.experimental.pallas import tpu as pltpu
from jax.experimental.pallas import tpu_sc as plsc


def _copy_body(x_ref, o_ref):
    o_ref[...] = x_ref[...]


def _tc_copy(x2d, blk):
    r, d = x2d.shape
    return pl.pallas_call(
        _copy_body,
        grid=(r // blk,),
        in_specs=[pl.BlockSpec((blk, d), lambda i: (i, 0))],
        out_specs=pl.BlockSpec((blk, d), lambda i: (i, 0)),
        out_shape=jax.ShapeDtypeStruct(x2d.shape, x2d.dtype),
    )(x2d)


def _make_sc_scatter(b, s, d, n, chunk=16):
    nc, ns = 2, 16
    mesh = plsc.VectorSubcoreMesh(
        core_axis_name="c", subcore_axis_name="s", num_cores=nc, num_subcores=ns
    )
    ntasks = b * n
    nworkers = ntasks // chunk

    @functools.partial(
        pl.kernel,
        out_type=(),
        mesh=mesh,
        scratch_types=[
            pltpu.VMEM((chunk,), jnp.int32),
            pltpu.VMEM((chunk, d), jnp.float32),
            pltpu.SemaphoreType.DMA,
        ],
    )
    def sc_scatter(out_ref, vals_hbm, idx_hbm, idx_v, rows_v, sem):
        wid = lax.axis_index("s") * nc + lax.axis_index("c")

        @pl.when(wid < nworkers)
        def _():
            t0 = wid * chunk
            batch = t0 // n
            i0 = t0 % n
            pltpu.sync_copy(idx_hbm.at[pl.ds(i0, chunk)], idx_v)
            pltpu.sync_copy(vals_hbm.at[pl.ds(i0, chunk)], rows_v)
            flat = idx_v[...] + batch * s
            pltpu.async_copy(rows_v, out_ref.at[flat], sem).wait()

    return sc_scatter


def kernel(x, replace_vals, replace_idx):
    b, s, d = x.shape
    n = replace_vals.shape[0]
    x2d = x.reshape(b * s, d)
    y = _tc_copy(x2d, blk=512)
    y_ref = jax.new_ref(y)
    _make_sc_scatter(b, s, d, n)(y_ref, replace_vals, replace_idx)
    return jax.freeze(y_ref).reshape(b, s, d)
